# baseline (speedup 1.0000x reference)
import functools

import jax
import jax.numpy as jnp
from jax import lax
from jax.experimental import pallas as pl
from jax.experimental.pallas import tpu as pltpu
from jax.experimental.pallas import tpu_sc as plsc

_EPS = 1e-5


@functools.cache
def _make_sc_gather(B, D, n_users, n_items):
    info = plsc.get_sparse_core_info()
    NC, NS = info.num_cores, info.num_subcores
    NW = NC * NS
    assert B % (8 * NW) == 0 and D % info.num_lanes == 0
    bpw = B // NW

    mesh = plsc.VectorSubcoreMesh(core_axis_name="c", subcore_axis_name="s")

    @functools.partial(
        pl.kernel,
        mesh=mesh,
        compiler_params=pltpu.CompilerParams(use_tc_tiling_on_sc=False),
        out_type=(
            jax.ShapeDtypeStruct((B, D), jnp.float32),
            jax.ShapeDtypeStruct((B, D), jnp.float32),
            jax.ShapeDtypeStruct((B, D), jnp.float32),
            jax.ShapeDtypeStruct((B, D), jnp.float32),
        ),
        scratch_types=[
            pltpu.VMEM((bpw,), jnp.int32),
            pltpu.VMEM((bpw,), jnp.int32),
            pltpu.VMEM((bpw, D), jnp.float32),
            pltpu.VMEM((bpw, D), jnp.float32),
            pltpu.SemaphoreType.DMA,
            pltpu.SemaphoreType.DMA,
        ],
    )
    def sc_gather(uid_hbm, iid_hbm, gu_hbm, gi_hbm, mu_hbm, mi_hbm,
                  o_gu, o_gi, o_mu, o_mi, uidx, iidx, bufa, bufb,
                  sem_a, sem_b):
        wid = lax.axis_index("s") * NC + lax.axis_index("c")
        base = wid * bpw
        pltpu.sync_copy(uid_hbm.at[pl.ds(base, bpw)], uidx)
        pltpu.sync_copy(iid_hbm.at[pl.ds(base, bpw)], iidx)
        c0 = pltpu.async_copy(gu_hbm.at[uidx], bufa, sem_a)
        c1 = pltpu.async_copy(gi_hbm.at[iidx], bufb, sem_b)
        c0.wait()
        pltpu.sync_copy(bufa, o_gu.at[pl.ds(base, bpw)])
        c2 = pltpu.async_copy(mu_hbm.at[uidx], bufa, sem_a)
        c1.wait()
        pltpu.sync_copy(bufb, o_gi.at[pl.ds(base, bpw)])
        c3 = pltpu.async_copy(mi_hbm.at[iidx], bufb, sem_b)
        c2.wait()
        pltpu.sync_copy(bufa, o_mu.at[pl.ds(base, bpw)])
        c3.wait()
        pltpu.sync_copy(bufb, o_mi.at[pl.ds(base, bpw)])

    return sc_gather


def _tc_body(gu, gi, mu, mi, w0a, w0b, b0, w1, b1, w2, b2, wg, wm, bo, out):
    f32 = jnp.float32
    h = jnp.dot(mu[...], w0a[...], preferred_element_type=f32)
    h = h + jnp.dot(mi[...], w0b[...], preferred_element_type=f32)
    h = jnp.maximum(h + b0[...], 0.0)
    h = jnp.maximum(jnp.dot(h, w1[...], preferred_element_type=f32) + b1[...], 0.0)
    h = jnp.maximum(jnp.dot(h, w2[...], preferred_element_type=f32) + b2[...], 0.0)
    g = gu[...] * gi[...]
    logit = (jnp.sum(g * wg[...], axis=1, keepdims=True)
             + jnp.sum(h * wm[...], axis=1, keepdims=True) + bo[...])
    out[...] = 1.0 / (1.0 + jnp.exp(-logit))


def _tc_dense(gu, gi, mu, mi, w0a, w0b, b0, w1, b1, w2, b2, wg, wm, bo2):
    B, D = gu.shape
    BLK = 2048
    H0, H1, H2 = w0a.shape[1], w1.shape[1], w2.shape[1]
    full = lambda s: pl.BlockSpec(s, lambda i: (0, 0))
    out = pl.pallas_call(
        _tc_body,
        grid=(B // BLK,),
        in_specs=[
            pl.BlockSpec((BLK, D), lambda i: (i, 0)),
            pl.BlockSpec((BLK, D), lambda i: (i, 0)),
            pl.BlockSpec((BLK, D), lambda i: (i, 0)),
            pl.BlockSpec((BLK, D), lambda i: (i, 0)),
            full((D, H0)), full((D, H0)), full((1, H0)),
            full((H0, H1)), full((1, H1)),
            full((H1, H2)), full((1, H2)),
            full((1, D)), full((1, H2)), full((1, 1)),
        ],
        out_specs=pl.BlockSpec((BLK, 1), lambda i: (i, 0)),
        out_shape=jax.ShapeDtypeStruct((B, 1), jnp.float32),
    )(gu, gi, mu, mi, w0a, w0b, b0, w1, b1, w2, b2, wg, wm, bo2)
    return out.reshape(B)


def kernel(user_ids, item_ids, gmf_user_w, gmf_item_w, mlp_user_w, mlp_item_w,
           W0, b0, g0, be0, W1, b1, g1, be1, W2, b2, g2, be2, Wo, bo):
    B = user_ids.shape[0]
    D = gmf_user_w.shape[1]
    uid = user_ids.astype(jnp.int32)
    iid = item_ids.astype(jnp.int32)

    gu, gi, mu, mi = _make_sc_gather(
        B, D, gmf_user_w.shape[0], gmf_item_w.shape[0])(
        uid, iid, gmf_user_w, gmf_item_w, mlp_user_w, mlp_item_w)

    inv = 1.0 / jnp.sqrt(jnp.float32(1.0 + _EPS))
    def fold(W, b, g, be):
        s = g * inv
        return (W * s[:, None]).T, (b * s + be)[None, :]
    w0f, b0f = fold(W0, b0, g0, be0)
    w1f, b1f = fold(W1, b1, g1, be1)
    w2f, b2f = fold(W2, b2, g2, be2)
    w0a, w0b = w0f[:D], w0f[D:]
    wg = Wo[:, :D]
    wm = Wo[:, D:]
    bo2 = bo[None, :]

    return _tc_dense(gu, gi, mu, mi, w0a, w0b, b0f, w1f, b1f, w2f, b2f,
                     wg, wm, bo2)


# speedup vs baseline: 3.0391x; 3.0391x over previous
import functools

import jax
import jax.numpy as jnp
from jax import lax
from jax.experimental import pallas as pl
from jax.experimental.pallas import tpu as pltpu
from jax.experimental.pallas import tpu_sc as plsc

_EPS = 1e-5


@functools.cache
def _make_sc_gather(B, D, V):
    info = plsc.get_sparse_core_info()
    NC, NS, L = info.num_cores, info.num_subcores, info.num_lanes
    NW = NC * NS
    FPW = 4 * D // NW
    WPT = D // FPW
    HALF = B // 2
    assert B % (2 * L) == 0 and 4 * D % NW == 0

    mesh = plsc.VectorSubcoreMesh(core_axis_name="c", subcore_axis_name="s")

    @functools.partial(
        pl.kernel,
        mesh=mesh,
        compiler_params=pltpu.CompilerParams(needs_layout_passes=False),
        out_type=tuple(
            jax.ShapeDtypeStruct((D, B), jnp.float32) for _ in range(4)),
        scratch_types=[
            pltpu.VMEM((V,), jnp.float32),
            pltpu.VMEM((B,), jnp.int32),
            pltpu.VMEM((HALF,), jnp.float32),
        ],
    )
    def sc_gather(uid, iid, t0, t1, t2, t3, o0, o1, o2, o3,
                  colbuf, idbuf, outbuf):
        wid = lax.axis_index("s") * NC + lax.axis_index("c")
        tbl = wid // WPT
        d0 = (wid % WPT) * FPW
        tables = ((t0, uid, o0), (t1, iid, o1), (t2, uid, o2), (t3, iid, o3))
        for t in range(4):
            tref, idsrc, oref = tables[t]

            @pl.when(tbl == t)
            def _():
                pltpu.sync_copy(idsrc, idbuf)
                for f in range(FPW):
                    d = d0 + f
                    pltpu.sync_copy(tref.at[d], colbuf)
                    for h in range(2):
                        @plsc.parallel_loop(0, HALF, step=L, unroll=8)
                        def _(i):
                            idx = idbuf[pl.ds(h * HALF + i, L)]
                            outbuf[pl.ds(i, L)] = plsc.load_gather(
                                colbuf, [idx])
                        pltpu.sync_copy(outbuf,
                                        oref.at[d, pl.ds(h * HALF, HALF)])

    return sc_gather


def _tc_body(guT, giT, muT, miT, w0a, w0b, b0, w1, b1, w2, b2, wg, wm, bo,
             out):
    f32 = jnp.float32
    h = jnp.dot(w0a[...], muT[...], preferred_element_type=f32)
    h = h + jnp.dot(w0b[...], miT[...], preferred_element_type=f32)
    h = jnp.maximum(h + b0[...], 0.0)
    h = jnp.maximum(
        jnp.dot(w1[...], h, preferred_element_type=f32) + b1[...], 0.0)
    h = jnp.maximum(
        jnp.dot(w2[...], h, preferred_element_type=f32) + b2[...], 0.0)
    g = guT[...] * giT[...]
    logit = (jnp.dot(wg[...], g, preferred_element_type=f32)
             + jnp.dot(wm[...], h, preferred_element_type=f32) + bo[...])
    out[...] = 1.0 / (1.0 + jnp.exp(-logit))


def _tc_dense(guT, giT, muT, miT, w0a, w0b, b0, w1, b1, w2, b2, wg, wm, bo2):
    D, B = guT.shape
    BLK = 2048
    H0, H1, H2 = w0a.shape[0], w1.shape[0], w2.shape[0]
    full = lambda s: pl.BlockSpec(s, lambda i: (0, 0))
    out = pl.pallas_call(
        _tc_body,
        grid=(B // BLK,),
        in_specs=[
            pl.BlockSpec((D, BLK), lambda i: (0, i)),
            pl.BlockSpec((D, BLK), lambda i: (0, i)),
            pl.BlockSpec((D, BLK), lambda i: (0, i)),
            pl.BlockSpec((D, BLK), lambda i: (0, i)),
            full((H0, D)), full((H0, D)), full((H0, 1)),
            full((H1, H0)), full((H1, 1)),
            full((H2, H1)), full((H2, 1)),
            full((1, D)), full((1, H2)), full((1, 1)),
        ],
        out_specs=pl.BlockSpec((1, BLK), lambda i: (0, i)),
        out_shape=jax.ShapeDtypeStruct((1, B), jnp.float32),
    )(guT, giT, muT, miT, w0a, w0b, b0, w1, b1, w2, b2, wg, wm, bo2)
    return out.reshape(B)


def kernel(user_ids, item_ids, gmf_user_w, gmf_item_w, mlp_user_w, mlp_item_w,
           W0, b0, g0, be0, W1, b1, g1, be1, W2, b2, g2, be2, Wo, bo):
    B = user_ids.shape[0]
    V, D = gmf_user_w.shape
    uid = user_ids.astype(jnp.int32)
    iid = item_ids.astype(jnp.int32)

    t0 = jnp.swapaxes(gmf_user_w, 0, 1)
    t1 = jnp.swapaxes(gmf_item_w, 0, 1)
    t2 = jnp.swapaxes(mlp_user_w, 0, 1)
    t3 = jnp.swapaxes(mlp_item_w, 0, 1)

    guT, giT, muT, miT = _make_sc_gather(B, D, V)(uid, iid, t0, t1, t2, t3)

    inv = 1.0 / jnp.sqrt(jnp.float32(1.0 + _EPS))
    def fold(W, b, g, be):
        s = g * inv
        return W * s[:, None], (b * s + be)[:, None]
    w0f, b0c = fold(W0, b0, g0, be0)
    w1f, b1c = fold(W1, b1, g1, be1)
    w2f, b2c = fold(W2, b2, g2, be2)
    w0a, w0b = w0f[:, :D], w0f[:, D:]
    wg = Wo[:, :D]
    wm = Wo[:, D:]
    bo2 = bo[None, :]

    return _tc_dense(guT, giT, muT, miT, w0a, w0b, b0c, w1f, b1c, w2f, b2c,
                     wg, wm, bo2)
